# baseline (speedup 1.0000x reference)
import functools

import jax
import jax.numpy as jnp
from jax.experimental import pallas as pl
from jax.experimental.pallas import tpu as pltpu

B = 4
S = 4096
D = 2048
K = 512


def _gather_mean_kernel(idx_ref, h_ref, o_ref):
    o_ref[...] = jnp.mean(h_ref[...], axis=0)


def _gather_mean(hidden_states, topk_indices):
    h4 = hidden_states.reshape(B, S, 16, 128)
    out = pl.pallas_call(
        _gather_mean_kernel,
        grid_spec=pltpu.PrefetchScalarGridSpec(
            num_scalar_prefetch=1,
            grid=(K,),
            in_specs=[
                pl.BlockSpec((B, 1, 16, 128), lambda i, idx_ref: (0, idx_ref[i], 0, 0)),
            ],
            out_specs=pl.BlockSpec((1, 16, 128), lambda i, idx_ref: (i, 0, 0)),
        ),
        out_shape=jax.ShapeDtypeStruct((K, 16, 128), jnp.float32),
    )(topk_indices, h4)
    return out.reshape(K, D)


def kernel(hidden_states, memory):
    importance = jnp.linalg.norm(hidden_states, axis=-1).mean(axis=0)
    _, topk_indices = jax.lax.top_k(importance, K)
    return _gather_mean(hidden_states, topk_indices)
